# Initial kernel scaffold; baseline (speedup 1.0000x reference)
import jax
import jax.numpy as jnp
from jax.experimental import pallas as pl


def kernel(color, music, task, color_table, music_table, task_table):
    raise NotImplementedError("write your pallas kernel here")


# speedup vs baseline: 29.2241x; 29.2241x over previous
import dataclasses
import functools

import jax
import jax.numpy as jnp
from jax import lax
from jax.experimental import pallas as pl
from jax.experimental.pallas import tpu as pltpu
from jax.experimental.pallas import tpu_sc as plsc

B, L, NROW = 16384, 200, 12
NC, NS = 2, 16
NW = NC * NS
ROWS_PER_TILE = B // NW
R = 16
NBLK = ROWS_PER_TILE // R
_IN_CH = R * L
_OUT_CH = R * L * NROW


def _sc_impl(ft, c_flat, m_flat, t_flat):
    mesh = plsc.VectorSubcoreMesh(core_axis_name="c", subcore_axis_name="s")
    cp = pltpu.CompilerParams()
    if "needs_layout_passes" in pltpu.CompilerParams.__dataclass_fields__:
        cp = dataclasses.replace(cp, needs_layout_passes=False)

    @functools.partial(
        pl.kernel,
        mesh=mesh,
        compiler_params=cp,
        out_type=jax.ShapeDtypeStruct((B * NROW * L,), jnp.float32),
        scratch_types=[
            pltpu.VMEM((NROW, 18), jnp.float32),
            pltpu.VMEM((_IN_CH,), jnp.int32),
            pltpu.VMEM((_IN_CH,), jnp.int32),
            pltpu.VMEM((_IN_CH,), jnp.int32),
            pltpu.VMEM((_IN_CH,), jnp.int32),
            pltpu.VMEM((_OUT_CH,), jnp.float32),
        ],
    )
    def k(ft_hbm, c_hbm, m_hbm, t_hbm, o_hbm, ft_vm, c_vm, m_vm, t_vm, f_vm, o_vm):
        wid = lax.axis_index("c") * NS + lax.axis_index("s")
        pltpu.sync_copy(ft_hbm, ft_vm)

        @pl.loop(0, NBLK)
        def _(blk):
            row0 = wid * ROWS_PER_TILE + blk * R
            pltpu.sync_copy(c_hbm.at[pl.ds(row0 * L, _IN_CH)], c_vm)
            pltpu.sync_copy(m_hbm.at[pl.ds(row0 * L, _IN_CH)], m_vm)
            pltpu.sync_copy(t_hbm.at[pl.ds(row0 * L, _IN_CH)], t_vm)

            @pl.loop(0, _IN_CH, step=16)
            def _(o):
                cv = c_vm[pl.ds(o, 16)]
                mv = m_vm[pl.ds(o, 16)]
                tv = t_vm[pl.ds(o, 16)]
                f_vm[pl.ds(o, 16)] = cv * 6 + mv * 2 + tv

            for j in range(NROW):
                col = ft_vm.at[j]

                @pl.loop(0, R)
                def _(b, j=j, col=col):
                    fbase = b * L
                    obase = b * (NROW * L) + j * L

                    @pl.loop(0, 192, step=16)
                    def _(kk, fbase=fbase, obase=obase, col=col):
                        fv = f_vm[pl.ds(fbase + kk, 16)]
                        o_vm[pl.ds(obase + kk, 16)] = plsc.load_gather(col, [fv])

                    fv = f_vm[pl.ds(fbase + 184, 16)]
                    o_vm[pl.ds(obase + 184, 16)] = plsc.load_gather(col, [fv])

            pltpu.sync_copy(o_vm, o_hbm.at[pl.ds(row0 * NROW * L, _OUT_CH)])

    return k(ft, c_flat, m_flat, t_flat)


def kernel(color, music, task, color_table, music_table, task_table):
    v = jnp.arange(18)
    F = jnp.concatenate(
        [color_table[v // 6], music_table[(v % 6) // 2], task_table[v % 2]],
        axis=1,
    )
    out = _sc_impl(
        F.T,
        color.astype(jnp.int32).reshape(-1),
        music.astype(jnp.int32).reshape(-1),
        task.astype(jnp.int32).reshape(-1),
    )
    return out.reshape(B, NROW, L)


# speedup vs baseline: 39.8991x; 1.3653x over previous
import dataclasses
import functools

import jax
import jax.numpy as jnp
from jax import lax
from jax.experimental import pallas as pl
from jax.experimental.pallas import tpu as pltpu
from jax.experimental.pallas import tpu_sc as plsc

B, L, NROW = 16384, 200, 12
NC, NS = 2, 16
NW = NC * NS
ROWS_PER_TILE = B // NW
R = 8
NBLK = ROWS_PER_TILE // R
_CHUNKS = tuple(range(0, 192, 16)) + (184,)


def _sc_impl(ft, c, m, t):
    mesh = plsc.VectorSubcoreMesh(core_axis_name="c", subcore_axis_name="s")
    cp = pltpu.CompilerParams()
    if "needs_layout_passes" in pltpu.CompilerParams.__dataclass_fields__:
        cp = dataclasses.replace(cp, needs_layout_passes=False)

    @functools.partial(
        pl.kernel,
        mesh=mesh,
        compiler_params=cp,
        out_type=jax.ShapeDtypeStruct((B, NROW, L), jnp.float32),
        scratch_types=[
            pltpu.VMEM((NROW, 18), jnp.float32),
            pltpu.VMEM((2, R, L), jnp.int32),
            pltpu.VMEM((2, R, L), jnp.int32),
            pltpu.VMEM((2, R, L), jnp.int32),
            pltpu.VMEM((R, L), jnp.int32),
            pltpu.VMEM((2, R, NROW, L), jnp.float32),
            pltpu.SemaphoreType.DMA((2,)),
            pltpu.SemaphoreType.DMA((2,)),
        ],
    )
    def k(ft_hbm, c_hbm, m_hbm, t_hbm, o_hbm,
          ft_vm, c_vm, m_vm, t_vm, f_vm, o_vm, sem_in, sem_out):
        wid = lax.axis_index("c") * NS + lax.axis_index("s")
        tile_row0 = wid * ROWS_PER_TILE
        pltpu.sync_copy(ft_hbm, ft_vm)

        def start_in(blk, buf):
            row0 = tile_row0 + blk * R
            for hbm, vm in ((c_hbm, c_vm), (m_hbm, m_vm), (t_hbm, t_vm)):
                pltpu.make_async_copy(
                    hbm.at[pl.ds(row0, R)], vm.at[buf], sem_in.at[buf]
                ).start()

        def wait_in(blk, buf):
            row0 = tile_row0 + blk * R
            for hbm, vm in ((c_hbm, c_vm), (m_hbm, m_vm), (t_hbm, t_vm)):
                pltpu.make_async_copy(
                    hbm.at[pl.ds(row0, R)], vm.at[buf], sem_in.at[buf]
                ).wait()

        def out_copy(blk, buf):
            row0 = tile_row0 + blk * R
            return pltpu.make_async_copy(
                o_hbm.at[pl.ds(row0, R)], o_vm.at[buf], sem_out.at[buf]
            )

        def start_out(blk, buf):
            row0 = tile_row0 + blk * R
            pltpu.make_async_copy(
                o_vm.at[buf], o_hbm.at[pl.ds(row0, R)], sem_out.at[buf]
            ).start()

        def wait_out(blk, buf):
            row0 = tile_row0 + blk * R
            pltpu.make_async_copy(
                o_vm.at[buf], o_hbm.at[pl.ds(row0, R)], sem_out.at[buf]
            ).wait()

        def compute(buf):
            @pl.loop(0, R)
            def _(b):
                for kk in _CHUNKS:
                    cv = c_vm[buf, b, pl.ds(kk, 16)]
                    mv = m_vm[buf, b, pl.ds(kk, 16)]
                    tv = t_vm[buf, b, pl.ds(kk, 16)]
                    f_vm[b, pl.ds(kk, 16)] = cv * 6 + mv * 2 + tv

            for j in range(NROW):
                col = ft_vm.at[j]

                @pl.loop(0, R)
                def _(b, j=j, col=col):
                    for kk in _CHUNKS:
                        fv = f_vm[b, pl.ds(kk, 16)]
                        o_vm[buf, b, j, pl.ds(kk, 16)] = plsc.load_gather(
                            col, [fv])

        start_in(0, 0)

        @pl.loop(0, NBLK, step=2)
        def _(i):
            for buf in (0, 1):
                blk = i + buf

                @pl.when(blk + 1 < NBLK)
                def _():
                    start_in(blk + 1, 1 - buf)

                wait_in(blk, buf)

                @pl.when(blk >= 2)
                def _():
                    wait_out(blk - 2, buf)

                compute(buf)
                start_out(blk, buf)

        wait_out(NBLK - 2, 0)
        wait_out(NBLK - 1, 1)

    return k(ft, c, m, t)


def kernel(color, music, task, color_table, music_table, task_table):
    v = jnp.arange(18)
    F = jnp.concatenate(
        [color_table[v // 6], music_table[(v % 6) // 2], task_table[v % 2]],
        axis=1,
    )
    return _sc_impl(
        F.T,
        color.astype(jnp.int32),
        music.astype(jnp.int32),
        task.astype(jnp.int32),
    )


# speedup vs baseline: 61.9062x; 1.5516x over previous
import dataclasses
import functools

import jax
import jax.numpy as jnp
from jax import lax
from jax.experimental import pallas as pl
from jax.experimental.pallas import tpu as pltpu
from jax.experimental.pallas import tpu_sc as plsc

B, L, NROW = 16384, 200, 12
NC, NS = 2, 16
NW = NC * NS
ROWS_PER_TILE = B // NW
R = 8
NBLK = ROWS_PER_TILE // R
_CHUNKS = tuple(range(0, 192, 16)) + (184,)


def _sc_impl(ft, c, m, t):
    mesh = plsc.VectorSubcoreMesh(core_axis_name="c", subcore_axis_name="s")
    cp = pltpu.CompilerParams()
    if "needs_layout_passes" in pltpu.CompilerParams.__dataclass_fields__:
        cp = dataclasses.replace(cp, needs_layout_passes=False)

    @functools.partial(
        pl.kernel,
        mesh=mesh,
        compiler_params=cp,
        out_type=jax.ShapeDtypeStruct((B, NROW, L), jnp.float32),
        scratch_types=[
            pltpu.VMEM((NROW, 18), jnp.float32),
            pltpu.VMEM((2, R, L), jnp.int32),
            pltpu.VMEM((2, R, L), jnp.int32),
            pltpu.VMEM((2, R, L), jnp.int32),
            pltpu.VMEM((2, R, NROW, L), jnp.float32),
            pltpu.SemaphoreType.DMA((2,)),
            pltpu.SemaphoreType.DMA((2,)),
        ],
    )
    def k(ft_hbm, c_hbm, m_hbm, t_hbm, o_hbm,
          ft_vm, c_vm, m_vm, t_vm, o_vm, sem_in, sem_out):
        wid = lax.axis_index("c") * NS + lax.axis_index("s")
        tile_row0 = wid * ROWS_PER_TILE
        pltpu.sync_copy(ft_hbm, ft_vm)

        def start_in(blk, buf):
            row0 = tile_row0 + blk * R
            for hbm, vm in ((c_hbm, c_vm), (m_hbm, m_vm), (t_hbm, t_vm)):
                pltpu.make_async_copy(
                    hbm.at[pl.ds(row0, R)], vm.at[buf], sem_in.at[buf]
                ).start()

        def wait_in(blk, buf):
            row0 = tile_row0 + blk * R
            for hbm, vm in ((c_hbm, c_vm), (m_hbm, m_vm), (t_hbm, t_vm)):
                pltpu.make_async_copy(
                    hbm.at[pl.ds(row0, R)], vm.at[buf], sem_in.at[buf]
                ).wait()

        def out_copy(blk, buf):
            row0 = tile_row0 + blk * R
            return pltpu.make_async_copy(
                o_hbm.at[pl.ds(row0, R)], o_vm.at[buf], sem_out.at[buf]
            )

        def start_out(blk, buf):
            row0 = tile_row0 + blk * R
            pltpu.make_async_copy(
                o_vm.at[buf], o_hbm.at[pl.ds(row0, R)], sem_out.at[buf]
            ).start()

        def wait_out(blk, buf):
            row0 = tile_row0 + blk * R
            pltpu.make_async_copy(
                o_vm.at[buf], o_hbm.at[pl.ds(row0, R)], sem_out.at[buf]
            ).wait()

        def compute(buf):
            @pl.loop(0, R)
            def _(b):
                for kk in _CHUNKS:
                    cv = c_vm[buf, b, pl.ds(kk, 16)]
                    mv = m_vm[buf, b, pl.ds(kk, 16)]
                    tv = t_vm[buf, b, pl.ds(kk, 16)]
                    fv = cv * 6 + mv * 2 + tv
                    for j in range(NROW):
                        o_vm[buf, b, j, pl.ds(kk, 16)] = plsc.load_gather(
                            ft_vm.at[j], [fv])

        start_in(0, 0)

        @pl.loop(0, NBLK, step=2)
        def _(i):
            for buf in (0, 1):
                blk = i + buf

                @pl.when(blk + 1 < NBLK)
                def _():
                    start_in(blk + 1, 1 - buf)

                wait_in(blk, buf)

                @pl.when(blk >= 2)
                def _():
                    wait_out(blk - 2, buf)

                compute(buf)
                start_out(blk, buf)

        wait_out(NBLK - 2, 0)
        wait_out(NBLK - 1, 1)

    return k(ft, c, m, t)


def kernel(color, music, task, color_table, music_table, task_table):
    v = jnp.arange(18)
    F = jnp.concatenate(
        [color_table[v // 6], music_table[(v % 6) // 2], task_table[v % 2]],
        axis=1,
    )
    return _sc_impl(
        F.T,
        color.astype(jnp.int32),
        music.astype(jnp.int32),
        task.astype(jnp.int32),
    )


# speedup vs baseline: 68.3132x; 1.1035x over previous
import dataclasses
import functools

import jax
import jax.numpy as jnp
from jax import lax
from jax.experimental import pallas as pl
from jax.experimental.pallas import tpu as pltpu
from jax.experimental.pallas import tpu_sc as plsc

B, L, NROW = 16384, 200, 12
NC, NS = 2, 16
NW = NC * NS
ROWS_PER_TILE = B // NW
R = 8
NBLK = ROWS_PER_TILE // R
_CHUNKS = tuple(range(0, 192, 16)) + (184,)


def _sc_impl(ft, c, m, t):
    mesh = plsc.VectorSubcoreMesh(core_axis_name="c", subcore_axis_name="s")
    cp = pltpu.CompilerParams()
    if "needs_layout_passes" in pltpu.CompilerParams.__dataclass_fields__:
        cp = dataclasses.replace(cp, needs_layout_passes=False)

    @functools.partial(
        pl.kernel,
        mesh=mesh,
        compiler_params=cp,
        out_type=jax.ShapeDtypeStruct((B, NROW, L), jnp.float32),
        scratch_types=[
            pltpu.VMEM((NROW, 18), jnp.float32),
            pltpu.VMEM((2, R, L), jnp.int32),
            pltpu.VMEM((2, R, L), jnp.int32),
            pltpu.VMEM((2, R, L), jnp.int32),
            pltpu.VMEM((2, R, NROW, L), jnp.float32),
            pltpu.SemaphoreType.DMA((2,)),
            pltpu.SemaphoreType.DMA((2,)),
        ],
    )
    def k(ft_hbm, c_hbm, m_hbm, t_hbm, o_hbm,
          ft_vm, c_vm, m_vm, t_vm, o_vm, sem_in, sem_out):
        wid = lax.axis_index("c") * NS + lax.axis_index("s")
        tile_row0 = wid * ROWS_PER_TILE
        pltpu.sync_copy(ft_hbm, ft_vm)

        def start_in(blk, buf):
            row0 = tile_row0 + blk * R
            for hbm, vm in ((c_hbm, c_vm), (m_hbm, m_vm), (t_hbm, t_vm)):
                pltpu.make_async_copy(
                    hbm.at[pl.ds(row0, R)], vm.at[buf], sem_in.at[buf]
                ).start()

        def wait_in(blk, buf):
            row0 = tile_row0 + blk * R
            for hbm, vm in ((c_hbm, c_vm), (m_hbm, m_vm), (t_hbm, t_vm)):
                pltpu.make_async_copy(
                    hbm.at[pl.ds(row0, R)], vm.at[buf], sem_in.at[buf]
                ).wait()

        def out_copy(blk, buf):
            row0 = tile_row0 + blk * R
            return pltpu.make_async_copy(
                o_hbm.at[pl.ds(row0, R)], o_vm.at[buf], sem_out.at[buf]
            )

        def start_out(blk, buf):
            row0 = tile_row0 + blk * R
            pltpu.make_async_copy(
                o_vm.at[buf], o_hbm.at[pl.ds(row0, R)], sem_out.at[buf]
            ).start()

        def wait_out(blk, buf):
            row0 = tile_row0 + blk * R
            pltpu.make_async_copy(
                o_vm.at[buf], o_hbm.at[pl.ds(row0, R)], sem_out.at[buf]
            ).wait()

        def compute(buf):
            @plsc.parallel_loop(0, R, 1, unroll=2)
            def _(b):
                for kk in _CHUNKS:
                    cv = c_vm[buf, b, pl.ds(kk, 16)]
                    mv = m_vm[buf, b, pl.ds(kk, 16)]
                    tv = t_vm[buf, b, pl.ds(kk, 16)]
                    fv = cv * 6 + mv * 2 + tv
                    for j in range(NROW):
                        o_vm[buf, b, j, pl.ds(kk, 16)] = plsc.load_gather(
                            ft_vm.at[j], [fv])

        start_in(0, 0)

        @pl.loop(0, NBLK, step=2)
        def _(i):
            for buf in (0, 1):
                blk = i + buf

                @pl.when(blk + 1 < NBLK)
                def _():
                    start_in(blk + 1, 1 - buf)

                wait_in(blk, buf)

                @pl.when(blk >= 2)
                def _():
                    wait_out(blk - 2, buf)

                compute(buf)
                start_out(blk, buf)

        wait_out(NBLK - 2, 0)
        wait_out(NBLK - 1, 1)

    return k(ft, c, m, t)


def kernel(color, music, task, color_table, music_table, task_table):
    v = jnp.arange(18)
    F = jnp.concatenate(
        [color_table[v // 6], music_table[(v % 6) // 2], task_table[v % 2]],
        axis=1,
    )
    return _sc_impl(
        F.T,
        color.astype(jnp.int32),
        music.astype(jnp.int32),
        task.astype(jnp.int32),
    )
